# baseline (speedup 1.0000x reference)
import functools

import jax
import jax.numpy as jnp
from jax import lax
from jax.experimental import pallas as pl
from jax.experimental.pallas import tpu as pltpu
from jax.experimental.pallas import tpu_sc as plsc

B = 16384
D = 32
NC = 2
NS = 16
NW = NC * NS
BPW = B // NW
L = 16
GROUPS = BPW // L


def _mf_body(uid_hbm, iid_hbm, ub_hbm, ib_hbm, ul_hbm, il_hbm, out_hbm,
             uid_v, iid_v, urows_v, irows_v, ubias_v, ibias_v, out_v,
             sem0, sem1, sem2, sem3):
    c = lax.axis_index("c")
    s = lax.axis_index("s")
    wid = s * NC + c
    base = wid * BPW

    pltpu.sync_copy(uid_hbm.at[pl.ds(base, BPW)], uid_v)
    pltpu.sync_copy(iid_hbm.at[pl.ds(base, BPW)], iid_v)

    cp0 = pltpu.async_copy(ul_hbm.at[uid_v], urows_v, sem0)
    cp1 = pltpu.async_copy(il_hbm.at[iid_v], irows_v, sem1)
    cp2 = pltpu.async_copy(ub_hbm.at[uid_v], ubias_v, sem2)
    cp3 = pltpu.async_copy(ib_hbm.at[iid_v], ibias_v, sem3)
    cp0.wait()
    cp1.wait()
    cp2.wait()
    cp3.wait()

    def group(g, carry):
        rows = g * L + lax.iota(jnp.int32, L)
        acc = plsc.load_gather(ubias_v, [rows])
        acc = acc + plsc.load_gather(ibias_v, [rows])
        for d in range(D):
            col = jnp.full((L,), d, jnp.int32)
            u = plsc.load_gather(urows_v, [rows, col])
            v = plsc.load_gather(irows_v, [rows, col])
            acc = acc + u * v
        out_v[pl.ds(g * L, L)] = 1.0 / (1.0 + jnp.exp(-acc))
        return carry

    lax.fori_loop(0, GROUPS, group, 0)

    pltpu.sync_copy(out_v, out_hbm.at[pl.ds(base, BPW)])


@jax.jit
def kernel(user_ids, item_ids, user_bias_emb, item_bias_emb,
           user_latent_emb, item_latent_emb):
    mesh = plsc.VectorSubcoreMesh(
        core_axis_name="c", subcore_axis_name="s",
        num_cores=NC, num_subcores=NS)
    mf = pl.kernel(
        _mf_body,
        out_type=jax.ShapeDtypeStruct((B,), jnp.float32),
        mesh=mesh,
        compiler_params=pltpu.CompilerParams(
            needs_layout_passes=False, use_tc_tiling_on_sc=False),
        scratch_types=[
            pltpu.VMEM((BPW,), jnp.int32),
            pltpu.VMEM((BPW,), jnp.int32),
            pltpu.VMEM((BPW, D), jnp.float32),
            pltpu.VMEM((BPW, D), jnp.float32),
            pltpu.VMEM((BPW,), jnp.float32),
            pltpu.VMEM((BPW,), jnp.float32),
            pltpu.VMEM((BPW,), jnp.float32),
            pltpu.SemaphoreType.DMA,
            pltpu.SemaphoreType.DMA,
            pltpu.SemaphoreType.DMA,
            pltpu.SemaphoreType.DMA,
        ],
    )
    return mf(user_ids.astype(jnp.int32), item_ids.astype(jnp.int32),
              user_bias_emb.reshape(-1), item_bias_emb.reshape(-1),
              user_latent_emb, item_latent_emb)


# speedup vs baseline: 2.6039x; 2.6039x over previous
import jax
import jax.numpy as jnp
from jax import lax
from jax.experimental import pallas as pl
from jax.experimental.pallas import tpu as pltpu
from jax.experimental.pallas import tpu_sc as plsc

B = 16384
D = 32
NC = 2
NS = 16
NW = NC * NS
BPW = B // NW
L = 16
GRP = 8
PPW = 2 * BPW


def _mf_body(uid_hbm, iid_hbm, uidp_hbm, iidp_hbm, ub_hbm, ib_hbm,
             ult_hbm, ilt_hbm, out_hbm,
             uid_v, iid_v, uidp_v, iidp_v, iota_v, ubuf_v, ibuf_v,
             ubias_v, ibias_v, out_v, sem, bsem):
    c = lax.axis_index("c")
    s = lax.axis_index("s")
    wid = s * NC + c
    base = wid * BPW
    pbase = wid * PPW

    pltpu.sync_copy(uid_hbm.at[pl.ds(base, BPW)], uid_v)
    pltpu.sync_copy(iid_hbm.at[pl.ds(base, BPW)], iid_v)
    pltpu.sync_copy(uidp_hbm.at[pl.ds(pbase, PPW)], uidp_v)
    pltpu.sync_copy(iidp_hbm.at[pl.ds(pbase, PPW)], iidp_v)

    cb0 = pltpu.async_copy(ub_hbm.at[uid_v], ubias_v, bsem)
    cb1 = pltpu.async_copy(ib_hbm.at[iid_v], ibias_v, bsem)

    iota_v[pl.ds(0, L)] = lax.iota(jnp.int32, L)
    iota_v[pl.ds(L, L)] = lax.iota(jnp.int32, L) + L

    dlo = lax.iota(jnp.int32, L)
    dhi = dlo + L

    def group(g, res):
        res = jnp.where((g % 2) == 0, jnp.zeros_like(res), res)
        chunk_u = uidp_v[pl.ds(g * L, L)]
        chunk_i = iidp_v[pl.ds(g * L, L)]
        rb_u = (chunk_u // 128) * 128
        rb_i = (chunk_i // 128) * 128
        lane_u = chunk_u % 128
        lane_i = chunk_i % 128
        copies = []
        for k in range(GRP):
            copies.append(pltpu.async_copy(
                ult_hbm.at[iota_v, pl.ds(pl.multiple_of(rb_u[k], 128), 128)], ubuf_v.at[k], sem))
            copies.append(pltpu.async_copy(
                ilt_hbm.at[iota_v, pl.ds(pl.multiple_of(rb_i[k], 128), 128)], ibuf_v.at[k], sem))
        for cp in copies:
            cp.wait()
        lane_sel = lax.iota(jnp.int32, L)
        half = (g % 2) * GRP
        for k in range(GRP):
            kvec = jnp.full((L,), k, jnp.int32)
            ulane = jnp.zeros((L,), jnp.int32) + lane_u[k]
            ilane = jnp.zeros((L,), jnp.int32) + lane_i[k]
            u0 = plsc.load_gather(ubuf_v, [kvec, dlo, ulane])
            u1 = plsc.load_gather(ubuf_v, [kvec, dhi, ulane])
            v0 = plsc.load_gather(ibuf_v, [kvec, dlo, ilane])
            v1 = plsc.load_gather(ibuf_v, [kvec, dhi, ilane])
            dot = jnp.sum(u0 * v0 + u1 * v1)
            res = jnp.where(lane_sel == (half + k), dot, res)
        @pl.when(g % 2 == 1)
        def _():
            off = (g // 2) * L
            acc = res + ubias_v[pl.ds(off, L)] + ibias_v[pl.ds(off, L)]
            out_v[pl.ds(off, L)] = 1.0 / (1.0 + jnp.exp(-acc))
        return res

    cb0.wait()
    cb1.wait()

    lax.fori_loop(0, PPW // L, group, jnp.zeros((L,), jnp.float32))

    pltpu.sync_copy(out_v, out_hbm.at[pl.ds(base, BPW)])


@jax.jit
def kernel(user_ids, item_ids, user_bias_emb, item_bias_emb,
           user_latent_emb, item_latent_emb):
    mesh = plsc.VectorSubcoreMesh(
        core_axis_name="c", subcore_axis_name="s",
        num_cores=NC, num_subcores=NS)
    mf = pl.kernel(
        _mf_body,
        out_type=jax.ShapeDtypeStruct((B,), jnp.float32),
        mesh=mesh,
        compiler_params=pltpu.CompilerParams(
            needs_layout_passes=False, use_tc_tiling_on_sc=True),
        scratch_types=[
            pltpu.VMEM((BPW,), jnp.int32),
            pltpu.VMEM((BPW,), jnp.int32),
            pltpu.VMEM((PPW,), jnp.int32),
            pltpu.VMEM((PPW,), jnp.int32),
            pltpu.VMEM((D,), jnp.int32),
            pltpu.VMEM((GRP, D, 128), jnp.float32),
            pltpu.VMEM((GRP, D, 128), jnp.float32),
            pltpu.VMEM((BPW,), jnp.float32),
            pltpu.VMEM((BPW,), jnp.float32),
            pltpu.VMEM((BPW,), jnp.float32),
            pltpu.SemaphoreType.DMA,
            pltpu.SemaphoreType.DMA,
        ],
    )
    uid = user_ids.astype(jnp.int32)
    iid = item_ids.astype(jnp.int32)
    uidp = jnp.pad(uid.reshape(-1, GRP), ((0, 0), (0, L - GRP))).reshape(-1)
    iidp = jnp.pad(iid.reshape(-1, GRP), ((0, 0), (0, L - GRP))).reshape(-1)
    return mf(uid, iid, uidp, iidp,
              user_bias_emb.reshape(-1), item_bias_emb.reshape(-1),
              user_latent_emb.T, item_latent_emb.T)
